# Initial kernel scaffold; baseline (speedup 1.0000x reference)
import jax
import jax.numpy as jnp
from jax.experimental import pallas as pl


def kernel(x, batch):
    raise NotImplementedError("write your pallas kernel here")


# speedup vs baseline: 12.1628x; 12.1628x over previous
import functools

import jax
import jax.numpy as jnp
from jax import lax
from jax.experimental import pallas as pl
from jax.experimental.pallas import tpu as pltpu
from jax.experimental.pallas import tpu_sc as plsc

N = 100000
D = 128
S = 512
L = 16
KS = D // L
NW = 32
SPW = S // NW
C = 128
NEG_INF = float("-inf")


def _sqrt(y):
    i = plsc.bitcast(y, jnp.int32)
    i = 0x5F3759DF - lax.shift_right_logical(i, 1)
    r = plsc.bitcast(i, jnp.float32)
    for _ in range(4):
        r = r * (1.5 - 0.5 * y * r * r)
    return y * r


def _sc_body(x_hbm, batch_hbm, out_hbm,
             idx_a, val_a, idx_b, val_b, bnd, xbuf, obuf, sem_a, sem_b):
    wid = lax.axis_index("s") * 2 + lax.axis_index("c")
    lane = lax.iota(jnp.int32, L)

    tgt_a = wid * SPW + lane
    tgt_b = jnp.full((L,), SPW, jnp.int32) + wid * SPW
    lo_a = jnp.zeros((L,), jnp.int32)
    hi_a = jnp.full((L,), N, jnp.int32)
    lo_b = jnp.zeros((L,), jnp.int32)
    hi_b = jnp.full((L,), N, jnp.int32)
    for _ in range(17):
        mid_a = lax.shift_right_logical(lo_a + hi_a, 1)
        mid_b = lax.shift_right_logical(lo_b + hi_b, 1)
        idx_a[...] = jnp.minimum(mid_a, N - 1)
        idx_b[...] = jnp.minimum(mid_b, N - 1)
        cp_a = pltpu.async_copy(batch_hbm.at[idx_a], val_a, sem_a)
        cp_b = pltpu.async_copy(batch_hbm.at[idx_b], val_b, sem_b)
        cp_a.wait()
        cp_b.wait()
        act_a = lo_a < hi_a
        act_b = lo_b < hi_b
        dn_a = act_a & (val_a[...] < tgt_a)
        dn_b = act_b & (val_b[...] < tgt_b)
        lo_a = jnp.where(dn_a, mid_a + 1, lo_a)
        hi_a = jnp.where(act_a & (~dn_a), mid_a, hi_a)
        lo_b = jnp.where(dn_b, mid_b + 1, lo_b)
        hi_b = jnp.where(act_b & (~dn_b), mid_b, hi_b)
    bnd[pl.ds(0, L)] = lo_a
    bnd[pl.ds(L, L)] = lo_b

    def seg_body(s_local, _):
        rs = bnd[pl.ds(s_local, L)][0]
        re = bnd[pl.ds(s_local + 1, L)][0]
        seg = wid * SPW + s_local

        zero = jnp.zeros((L,), jnp.float32)
        ninf = jnp.full((L,), NEG_INF, jnp.float32)
        acc0 = (tuple(zero for _ in range(KS)),
                tuple(zero for _ in range(KS)),
                tuple(ninf for _ in range(KS)))

        rs_al = jnp.bitwise_and(rs, -8)
        nch = lax.shift_right_logical(re - rs_al + (C - 1), 7)

        def chunk_body(c, carry):
            p0 = rs_al + c * C
            p = pl.multiple_of(jnp.minimum(p0, N - C), 8)
            lo_r = jnp.maximum(rs, p0) - p
            hi_r = jnp.minimum(re, p0 + C) - p
            pltpu.sync_copy(x_hbm.at[pl.ds(p, C)], xbuf)

            def row_body(r, rc):
                sums, sqs, mxs = rc
                ns, nq, nm = [], [], []
                for k in range(KS):
                    v = xbuf[r, pl.ds(k * L, L)]
                    ns.append(sums[k] + v)
                    nq.append(sqs[k] + v * v)
                    nm.append(jnp.maximum(mxs[k], v))
                return (tuple(ns), tuple(nq), tuple(nm))

            return lax.fori_loop(lo_r, hi_r, row_body, carry)

        sums, sqs, mxs = lax.fori_loop(0, nch, chunk_body, acc0)

        cntf = (re - rs).astype(jnp.float32)
        cnt_v = jnp.zeros((L,), jnp.float32) + cntf
        inv = 1.0 / jnp.maximum(cnt_v, 1.0)
        for k in range(KS):
            mean = sums[k] * inv
            var = jnp.maximum(sqs[k] * inv - mean * mean, 0.0)
            std = _sqrt(var + 1e-6)
            obuf[pl.ds(k * L, L)] = mean
            obuf[pl.ds(D + k * L, L)] = mxs[k]
            obuf[pl.ds(2 * D + k * L, L)] = sums[k]
            obuf[pl.ds(3 * D + k * L, L)] = std
        pltpu.sync_copy(obuf, out_hbm.at[pl.ds(seg * (4 * D), 4 * D)])
        return 0

    lax.fori_loop(0, SPW, seg_body, 0)


@jax.jit
def kernel(x, batch):
    batch_i32 = batch.astype(jnp.int32)
    mesh = plsc.VectorSubcoreMesh(core_axis_name="c", subcore_axis_name="s")
    f = pl.kernel(
        _sc_body,
        out_type=jax.ShapeDtypeStruct((S * 4 * D,), jnp.float32),
        mesh=mesh,
        compiler_params=pltpu.CompilerParams(needs_layout_passes=False),
        scratch_types=[
            pltpu.VMEM((L,), jnp.int32),
            pltpu.VMEM((L,), jnp.int32),
            pltpu.VMEM((L,), jnp.int32),
            pltpu.VMEM((L,), jnp.int32),
            pltpu.VMEM((2 * L,), jnp.int32),
            pltpu.VMEM((C, D), jnp.float32),
            pltpu.VMEM((4 * D,), jnp.float32),
            pltpu.SemaphoreType.DMA,
            pltpu.SemaphoreType.DMA,
        ],
    )
    return f(x, batch_i32).reshape(S, 4 * D)


# speedup vs baseline: 13.9273x; 1.1451x over previous
import functools

import jax
import jax.numpy as jnp
from jax import lax
from jax.experimental import pallas as pl
from jax.experimental.pallas import tpu as pltpu
from jax.experimental.pallas import tpu_sc as plsc

N = 100000
D = 128
S = 512
L = 16
KS = D // L
NW = 32
SPW = S // NW
C = 128
NEG_INF = float("-inf")


def _sqrt(y):
    i = plsc.bitcast(y, jnp.int32)
    i = 0x5F3759DF - lax.shift_right_logical(i, 1)
    r = plsc.bitcast(i, jnp.float32)
    for _ in range(4):
        r = r * (1.5 - 0.5 * y * r * r)
    return y * r


def _sc_body(x_hbm, batch_hbm, out_hbm,
             idx_a, val_a, idx_b, val_b, bnd, xbuf, obuf, sem_a, sem_b):
    wid = lax.axis_index("s") * 2 + lax.axis_index("c")
    lane = lax.iota(jnp.int32, L)

    tgt_a = wid * SPW + lane
    tgt_b = jnp.full((L,), SPW, jnp.int32) + wid * SPW
    lo_a = jnp.zeros((L,), jnp.int32)
    hi_a = jnp.full((L,), N, jnp.int32)
    lo_b = jnp.zeros((L,), jnp.int32)
    hi_b = jnp.full((L,), N, jnp.int32)
    for _ in range(17):
        mid_a = lax.shift_right_logical(lo_a + hi_a, 1)
        mid_b = lax.shift_right_logical(lo_b + hi_b, 1)
        idx_a[...] = jnp.minimum(mid_a, N - 1)
        idx_b[...] = jnp.minimum(mid_b, N - 1)
        cp_a = pltpu.async_copy(batch_hbm.at[idx_a], val_a, sem_a)
        cp_b = pltpu.async_copy(batch_hbm.at[idx_b], val_b, sem_b)
        cp_a.wait()
        cp_b.wait()
        act_a = lo_a < hi_a
        act_b = lo_b < hi_b
        dn_a = act_a & (val_a[...] < tgt_a)
        dn_b = act_b & (val_b[...] < tgt_b)
        lo_a = jnp.where(dn_a, mid_a + 1, lo_a)
        hi_a = jnp.where(act_a & (~dn_a), mid_a, hi_a)
        lo_b = jnp.where(dn_b, mid_b + 1, lo_b)
        hi_b = jnp.where(act_b & (~dn_b), mid_b, hi_b)
    bnd[pl.ds(0, L)] = lo_a
    bnd[pl.ds(L, L)] = lo_b

    def seg_body(s_local, _):
        rs = bnd[pl.ds(s_local, L)][0]
        re = bnd[pl.ds(s_local + 1, L)][0]

        zero = jnp.zeros((L,), jnp.float32)
        ninf = jnp.full((L,), NEG_INF, jnp.float32)
        acc0 = (tuple(zero for _ in range(KS)),
                tuple(zero for _ in range(KS)),
                tuple(ninf for _ in range(KS)))

        rs_al = jnp.bitwise_and(rs, -8)
        nch = lax.shift_right_logical(re - rs_al + (C - 1), 7)

        def p_of(c):
            return pl.multiple_of(jnp.minimum(rs_al + c * C, N - C), 8)

        def start_dma(c):
            p = p_of(c)

            @pl.when(jnp.bitwise_and(c, 1) == 0)
            def _():
                pltpu.async_copy(x_hbm.at[pl.ds(p, C)], xbuf.at[0], sem_a)

            @pl.when(jnp.bitwise_and(c, 1) == 1)
            def _():
                pltpu.async_copy(x_hbm.at[pl.ds(p, C)], xbuf.at[1], sem_b)

        def wait_dma(c):
            p = p_of(c)

            @pl.when(jnp.bitwise_and(c, 1) == 0)
            def _():
                pltpu.make_async_copy(
                    x_hbm.at[pl.ds(p, C)], xbuf.at[0], sem_a).wait()

            @pl.when(jnp.bitwise_and(c, 1) == 1)
            def _():
                pltpu.make_async_copy(
                    x_hbm.at[pl.ds(p, C)], xbuf.at[1], sem_b).wait()

        @pl.when(nch > 0)
        def _():
            start_dma(jnp.int32(0))

        def chunk_body(c, carry):
            p0 = rs_al + c * C
            p = p_of(c)
            lo_r = jnp.maximum(rs, p0) - p
            hi_r = jnp.minimum(re, p0 + C) - p
            par = jnp.bitwise_and(c, 1)
            wait_dma(c)

            @pl.when(c + 1 < nch)
            def _():
                start_dma(c + 1)

            def acc_row(r, rc):
                sums, sqs, mxs = rc
                ns, nq, nm = [], [], []
                for k in range(KS):
                    v = xbuf[par, r, pl.ds(k * L, L)]
                    ns.append(sums[k] + v)
                    nq.append(sqs[k] + v * v)
                    nm.append(jnp.maximum(mxs[k], v))
                return (tuple(ns), tuple(nq), tuple(nm))

            n4 = lax.shift_right_logical(hi_r - lo_r, 2)

            def row_body4(i, rc):
                r = lo_r + i * 4
                for u in range(4):
                    rc = acc_row(r + u, rc)
                return rc

            carry = lax.fori_loop(0, n4, row_body4, carry)
            return lax.fori_loop(lo_r + n4 * 4, hi_r, acc_row, carry)

        sums, sqs, mxs = lax.fori_loop(0, nch, chunk_body, acc0)

        cntf = (re - rs).astype(jnp.float32)
        cnt_v = jnp.zeros((L,), jnp.float32) + cntf
        inv = 1.0 / jnp.maximum(cnt_v, 1.0)
        ob = s_local * (4 * D)
        for k in range(KS):
            mean = sums[k] * inv
            var = jnp.maximum(sqs[k] * inv - mean * mean, 0.0)
            std = _sqrt(var + 1e-6)
            obuf[pl.ds(ob + k * L, L)] = mean
            obuf[pl.ds(ob + D + k * L, L)] = mxs[k]
            obuf[pl.ds(ob + 2 * D + k * L, L)] = sums[k]
            obuf[pl.ds(ob + 3 * D + k * L, L)] = std
        return 0

    lax.fori_loop(0, SPW, seg_body, 0)
    pltpu.sync_copy(obuf, out_hbm.at[pl.ds(wid * SPW * 4 * D, SPW * 4 * D)])


@jax.jit
def kernel(x, batch):
    batch_i32 = batch.astype(jnp.int32)
    mesh = plsc.VectorSubcoreMesh(core_axis_name="c", subcore_axis_name="s")
    f = pl.kernel(
        _sc_body,
        out_type=jax.ShapeDtypeStruct((S * 4 * D,), jnp.float32),
        mesh=mesh,
        compiler_params=pltpu.CompilerParams(needs_layout_passes=False),
        scratch_types=[
            pltpu.VMEM((L,), jnp.int32),
            pltpu.VMEM((L,), jnp.int32),
            pltpu.VMEM((L,), jnp.int32),
            pltpu.VMEM((L,), jnp.int32),
            pltpu.VMEM((2 * L,), jnp.int32),
            pltpu.VMEM((2, C, D), jnp.float32),
            pltpu.VMEM((SPW * 4 * D,), jnp.float32),
            pltpu.SemaphoreType.DMA,
            pltpu.SemaphoreType.DMA,
        ],
    )
    return f(x, batch_i32).reshape(S, 4 * D)
